# baseline (speedup 1.0000x reference)
import jax
import jax.numpy as jnp
from jax import lax
from jax.experimental import pallas as pl
from jax.experimental.pallas import tpu as pltpu
from jax.experimental.pallas import tpu_sc as plsc

_PERM = [0, 2, 4, 6, 8, 10, 12, 1, 3, 5, 7, 9, 11, 13]
_CYCLE = [1, 2, 4, 8, 3, 6, 12, 11, 9, 5, 10, 7]

_N_IMG = 8 * 192
_H = 224
_N_WORKERS = 32
_IMG_PER_W = _N_IMG // _N_WORKERS
_TASKS = 2 * _IMG_PER_W
_RING = 4


def _sc_body(x_hbm, out_hbm, ib0, ib1, ib2, ib3, r0, r1, r2, r3, w0, w1, w2, w3):
    wid = lax.axis_index("s") * 2 + lax.axis_index("c")
    img0 = wid * _IMG_PER_W
    bufs = (ib0, ib1, ib2, ib3)
    rsems = (r0, r1, r2, r3)
    wsems = (w0, w1, w2, w3)

    def start_read(t, slot):
        img = img0 + t // 2
        i2 = t % 2
        for j7 in range(7):
            pltpu.async_copy(
                x_hbm.at[img, pl.ds(32 * j7 + 16 * i2, 16)],
                bufs[slot].at[pl.ds(16 * j7, 16)],
                rsems[slot],
            )

    def wait_read(slot):
        pltpu.make_async_copy(
            x_hbm.at[img0, pl.ds(0, 112)], bufs[slot], rsems[slot]
        ).wait()

    def wait_writes(slot):
        pltpu.make_async_copy(
            bufs[slot], out_hbm.at[img0, pl.ds(0, 112)], wsems[slot]
        ).wait()

    def shuffle_chunk(buf, j7):
        @pl.loop(16 * j7, 16 * j7 + 16, unroll=16)
        def _row(r):
            tmp = buf[r, pl.ds(16 * _CYCLE[0], 16)]
            for i in range(len(_CYCLE) - 1):
                buf[r, pl.ds(16 * _CYCLE[i], 16)] = (
                    buf[r, pl.ds(16 * _CYCLE[i + 1], 16)]
                )
            buf[r, pl.ds(16 * _CYCLE[-1], 16)] = tmp

    def process(t, slot):
        img = img0 + t // 2
        i2 = t % 2
        for j7 in range(7):
            shuffle_chunk(bufs[slot], j7)
            pltpu.async_copy(
                bufs[slot].at[pl.ds(16 * j7, 16)],
                out_hbm.at[img, pl.ds(112 * i2 + 16 * j7, 16)],
                wsems[slot],
            )

    start_read(0, 0)
    start_read(1, 1)

    @pl.loop(0, _TASKS, step=_RING)
    def _pipe(g):
        for b in range(_RING):
            t = g + b
            nxt = (b + 2) % _RING

            @pl.when(t >= 2)
            def _():
                wait_writes(nxt)

            @pl.when(t + 2 < _TASKS)
            def _():
                start_read(t + 2, nxt)

            wait_read(b)
            process(t, b)

    wait_writes((_TASKS - 2) % _RING)
    wait_writes((_TASKS - 1) % _RING)


def kernel(x):
    x3 = x.reshape(_N_IMG, _H, _H)
    mesh = plsc.VectorSubcoreMesh(core_axis_name="c", subcore_axis_name="s")
    run = pl.kernel(
        _sc_body,
        out_type=jax.ShapeDtypeStruct((_N_IMG, _H, _H), jnp.float32),
        mesh=mesh,
        scratch_types=[
            pltpu.VMEM((112, _H), jnp.float32),
            pltpu.VMEM((112, _H), jnp.float32),
            pltpu.VMEM((112, _H), jnp.float32),
            pltpu.VMEM((112, _H), jnp.float32),
            pltpu.SemaphoreType.DMA,
            pltpu.SemaphoreType.DMA,
            pltpu.SemaphoreType.DMA,
            pltpu.SemaphoreType.DMA,
            pltpu.SemaphoreType.DMA,
            pltpu.SemaphoreType.DMA,
            pltpu.SemaphoreType.DMA,
            pltpu.SemaphoreType.DMA,
        ],
    )
    y = run(x3)
    return y.reshape(x.shape)


# speedup vs baseline: 1.0328x; 1.0328x over previous
import jax
import jax.numpy as jnp
from jax import lax
from jax.experimental import pallas as pl
from jax.experimental.pallas import tpu as pltpu
from jax.experimental.pallas import tpu_sc as plsc

_PERM = [0, 2, 4, 6, 8, 10, 12, 1, 3, 5, 7, 9, 11, 13]
_CYCLE = [1, 2, 4, 8, 3, 6, 12, 11, 9, 5, 10, 7]

_N_IMG = 8 * 192
_H = 224
_N_WORKERS = 32
_IMG_PER_W = _N_IMG // _N_WORKERS
_TASKS = 2 * _IMG_PER_W
_RING = 4


def _sc_body(x_hbm, out_hbm, ib0, ib1, ib2, ib3, r0, r1, r2, r3, w0, w1, w2, w3):
    wid = lax.axis_index("s") * 2 + lax.axis_index("c")
    img0 = wid * _IMG_PER_W
    bufs = (ib0, ib1, ib2, ib3)
    rsems = (r0, r1, r2, r3)
    wsems = (w0, w1, w2, w3)

    def start_read(t, slot):
        img = img0 + t // 2
        i2 = t % 2
        for j7 in range(7):
            pltpu.async_copy(
                x_hbm.at[img, pl.ds(32 * j7 + 16 * i2, 16)],
                bufs[slot].at[pl.ds(16 * j7, 16)],
                rsems[slot],
            )

    def wait_read(slot):
        pltpu.make_async_copy(
            x_hbm.at[img0, pl.ds(0, 112)], bufs[slot], rsems[slot]
        ).wait()

    def wait_writes(slot):
        pltpu.make_async_copy(
            bufs[slot], out_hbm.at[img0, pl.ds(0, 112)], wsems[slot]
        ).wait()

    def shuffle_chunk(buf, j7):
        @plsc.parallel_loop(16 * j7, 16 * j7 + 16, unroll=8)
        def _row(r):
            tmp = buf[r, pl.ds(16 * _CYCLE[0], 16)]
            for i in range(len(_CYCLE) - 1):
                buf[r, pl.ds(16 * _CYCLE[i], 16)] = (
                    buf[r, pl.ds(16 * _CYCLE[i + 1], 16)]
                )
            buf[r, pl.ds(16 * _CYCLE[-1], 16)] = tmp

    def process(t, slot):
        img = img0 + t // 2
        i2 = t % 2
        for j7 in range(7):
            shuffle_chunk(bufs[slot], j7)
            pltpu.async_copy(
                bufs[slot].at[pl.ds(16 * j7, 16)],
                out_hbm.at[img, pl.ds(112 * i2 + 16 * j7, 16)],
                wsems[slot],
            )

    start_read(0, 0)
    start_read(1, 1)

    @pl.loop(0, _TASKS, step=_RING)
    def _pipe(g):
        for b in range(_RING):
            t = g + b
            nxt = (b + 2) % _RING

            @pl.when(t >= 2)
            def _():
                wait_writes(nxt)

            @pl.when(t + 2 < _TASKS)
            def _():
                start_read(t + 2, nxt)

            wait_read(b)
            process(t, b)

    wait_writes((_TASKS - 2) % _RING)
    wait_writes((_TASKS - 1) % _RING)


def kernel(x):
    x3 = x.reshape(_N_IMG, _H, _H)
    mesh = plsc.VectorSubcoreMesh(core_axis_name="c", subcore_axis_name="s")
    run = pl.kernel(
        _sc_body,
        out_type=jax.ShapeDtypeStruct((_N_IMG, _H, _H), jnp.float32),
        mesh=mesh,
        scratch_types=[
            pltpu.VMEM((112, _H), jnp.float32),
            pltpu.VMEM((112, _H), jnp.float32),
            pltpu.VMEM((112, _H), jnp.float32),
            pltpu.VMEM((112, _H), jnp.float32),
            pltpu.SemaphoreType.DMA,
            pltpu.SemaphoreType.DMA,
            pltpu.SemaphoreType.DMA,
            pltpu.SemaphoreType.DMA,
            pltpu.SemaphoreType.DMA,
            pltpu.SemaphoreType.DMA,
            pltpu.SemaphoreType.DMA,
            pltpu.SemaphoreType.DMA,
        ],
    )
    y = run(x3)
    return y.reshape(x.shape)


# speedup vs baseline: 1.0948x; 1.0600x over previous
import jax
import jax.numpy as jnp
from jax import lax
from jax.experimental import pallas as pl
from jax.experimental.pallas import tpu as pltpu
from jax.experimental.pallas import tpu_sc as plsc

_PERM = [0, 2, 4, 6, 8, 10, 12, 1, 3, 5, 7, 9, 11, 13]
_CYCLE = [1, 2, 4, 8, 3, 6, 12, 11, 9, 5, 10, 7]

_N_IMG = 8 * 192
_H = 224
_N_WORKERS = 32
_IMG_PER_W = _N_IMG // _N_WORKERS
_TASKS = 2 * _IMG_PER_W
_RING = 4


def _sc_body(x_hbm, out_hbm, ib0, ib1, ib2, ib3, r0, r1, r2, r3, w0, w1, w2, w3):
    wid = lax.axis_index("s") * 2 + lax.axis_index("c")
    img0 = wid * _IMG_PER_W
    bufs = (ib0, ib1, ib2, ib3)
    rsems = (r0, r1, r2, r3)
    wsems = (w0, w1, w2, w3)

    def start_read(t, slot):
        img = img0 + t // 2
        i2 = t % 2
        for j7 in range(7):
            pltpu.async_copy(
                x_hbm.at[img, pl.ds(32 * j7 + 16 * i2, 16)],
                bufs[slot].at[pl.ds(16 * j7, 16)],
                rsems[slot],
            )

    def wait_read(slot):
        pltpu.make_async_copy(
            x_hbm.at[img0, pl.ds(0, 112)], bufs[slot], rsems[slot]
        ).wait()

    def wait_writes(slot):
        pltpu.make_async_copy(
            bufs[slot], out_hbm.at[img0, pl.ds(0, 112)], wsems[slot]
        ).wait()

    def shuffle_chunk(buf, j7):
        @pl.loop(16 * j7, 16 * j7 + 16, unroll=8)
        def _row(r):
            tmp = buf[r, pl.ds(16 * _CYCLE[0], 16)]
            for i in range(len(_CYCLE) - 1):
                buf[r, pl.ds(16 * _CYCLE[i], 16)] = (
                    buf[r, pl.ds(16 * _CYCLE[i + 1], 16)]
                )
            buf[r, pl.ds(16 * _CYCLE[-1], 16)] = tmp

    def process(t, slot):
        img = img0 + t // 2
        i2 = t % 2
        for j7 in range(7):
            shuffle_chunk(bufs[slot], j7)
            pltpu.async_copy(
                bufs[slot].at[pl.ds(16 * j7, 16)],
                out_hbm.at[img, pl.ds(112 * i2 + 16 * j7, 16)],
                wsems[slot],
            )

    start_read(0, 0)
    start_read(1, 1)

    @pl.loop(0, _TASKS, step=_RING)
    def _pipe(g):
        for b in range(_RING):
            t = g + b
            nxt = (b + 2) % _RING

            @pl.when(t >= 2)
            def _():
                wait_writes(nxt)

            @pl.when(t + 2 < _TASKS)
            def _():
                start_read(t + 2, nxt)

            wait_read(b)
            process(t, b)

    wait_writes((_TASKS - 2) % _RING)
    wait_writes((_TASKS - 1) % _RING)


def kernel(x):
    x3 = x.reshape(_N_IMG, _H, _H)
    mesh = plsc.VectorSubcoreMesh(core_axis_name="c", subcore_axis_name="s")
    run = pl.kernel(
        _sc_body,
        out_type=jax.ShapeDtypeStruct((_N_IMG, _H, _H), jnp.float32),
        mesh=mesh,
        scratch_types=[
            pltpu.VMEM((112, _H), jnp.float32),
            pltpu.VMEM((112, _H), jnp.float32),
            pltpu.VMEM((112, _H), jnp.float32),
            pltpu.VMEM((112, _H), jnp.float32),
            pltpu.SemaphoreType.DMA,
            pltpu.SemaphoreType.DMA,
            pltpu.SemaphoreType.DMA,
            pltpu.SemaphoreType.DMA,
            pltpu.SemaphoreType.DMA,
            pltpu.SemaphoreType.DMA,
            pltpu.SemaphoreType.DMA,
            pltpu.SemaphoreType.DMA,
        ],
    )
    y = run(x3)
    return y.reshape(x.shape)
